# baseline (speedup 1.0000x reference)
import functools

import jax
import jax.numpy as jnp
from jax import lax
from jax.experimental import pallas as pl
from jax.experimental.pallas import tpu as pltpu
from jax.experimental.pallas import tpu_sc as plsc

N = 10000
E = 320000
D = 128

NC = 2
NS = 16
NW = NC * NS

K = 128
CH = 80
EPAD = NW * CH * K
NPAD = 10240
RPT = NPAD // NS
ZB = K


def _sc_agg_body(h_hbm, src_hbm, dst_hbm, out_hbm,
                 idx_src, idx_dst, rows0, pooled, sem0):
    cid = lax.axis_index("c")
    sid = lax.axis_index("s")
    wid = sid * NC + cid

    def _zrow(i, _):
        for j in range(D // 16):
            rows0[i, pl.ds(j * 16, 16)] = jnp.zeros((16,), jnp.float32)
        return 0
    lax.fori_loop(0, ZB, _zrow, 0)
    for t in range(RPT // ZB):
        pltpu.sync_copy(rows0, pooled.at[pl.ds(sid * RPT + t * ZB, ZB)])
    plsc.subcore_barrier()

    pltpu.sync_copy(src_hbm.at[wid], idx_src)
    pltpu.sync_copy(dst_hbm.at[wid], idx_dst)

    def _chunk(j, _):
        pltpu.async_copy(h_hbm.at[idx_src.at[j]], rows0, sem0).wait()
        pltpu.sync_copy(rows0, pooled.at[idx_dst.at[j]], add=True)
        return 0
    lax.fori_loop(0, CH, _chunk, 0)
    plsc.subcore_barrier()

    for t in range(RPT // ZB):
        r0 = sid * RPT + t * ZB
        pltpu.sync_copy(pooled.at[pl.ds(r0, ZB)], rows0)
        pltpu.sync_copy(rows0, out_hbm.at[cid, pl.ds(r0, ZB)])


@functools.lru_cache(maxsize=None)
def _make_sc_agg():
    return pl.kernel(
        _sc_agg_body,
        out_type=jax.ShapeDtypeStruct((NC, NPAD, D), jnp.float32),
        mesh=plsc.VectorSubcoreMesh(core_axis_name="c", subcore_axis_name="s",
                                    num_cores=NC, num_subcores=NS),
        scratch_types=[
            pltpu.VMEM((CH, K), jnp.int32),
            pltpu.VMEM((CH, K), jnp.int32),
            pltpu.VMEM((K, D), jnp.float32),
            pltpu.VMEM_SHARED((NPAD, D), jnp.float32),
            pltpu.SemaphoreType.DMA,
        ],
    )


def _sc_agg(h, srcp, dstp):
    return _make_sc_agg()(h, srcp, dstp)


BM = 1000


def _dense_body(s_ref, h_ref, p_ref, w_ref, b_ref, o_ref):
    t = s_ref[0] * h_ref[...] + jnp.sum(p_ref[...], axis=0)
    o_ref[...] = jnp.maximum(
        jnp.dot(t, w_ref[...], preferred_element_type=jnp.float32)
        + b_ref[...], 0.0)


def _tc_dense(h, p, w, b, scale):
    return pl.pallas_call(
        _dense_body,
        grid=(N // BM,),
        in_specs=[
            pl.BlockSpec(memory_space=pltpu.SMEM),
            pl.BlockSpec((BM, D), lambda i: (i, 0)),
            pl.BlockSpec((NC, BM, D), lambda i: (0, i, 0)),
            pl.BlockSpec((D, D), lambda i: (0, 0)),
            pl.BlockSpec((1, D), lambda i: (0, 0)),
        ],
        out_specs=pl.BlockSpec((BM, D), lambda i: (i, 0)),
        out_shape=jax.ShapeDtypeStruct((N, D), jnp.float32),
    )(scale, h, p, w, b)


def _final_body(s_ref, h_ref, p_ref, w_ref, b_ref,
                fw1_ref, fb1_ref, fw2_ref, fb2_ref, o_ref, acc):
    i = pl.program_id(0)

    @pl.when(i == 0)
    def _():
        acc[...] = jnp.zeros_like(acc)

    t = s_ref[0] * h_ref[...] + jnp.sum(p_ref[...], axis=0)
    t = jnp.maximum(
        jnp.dot(t, w_ref[...], preferred_element_type=jnp.float32)
        + b_ref[...], 0.0)
    acc[...] += jnp.sum(t, axis=0, keepdims=True)

    @pl.when(i == pl.num_programs(0) - 1)
    def _():
        g = acc[...]
        t1 = jnp.maximum(
            jnp.dot(g, fw1_ref[...], preferred_element_type=jnp.float32)
            + fb1_ref[...], 0.0)
        logits = (jnp.dot(t1, fw2_ref[...], preferred_element_type=jnp.float32)
                  + fb2_ref[...])
        m = jnp.max(logits, axis=1, keepdims=True)
        e = jnp.exp(logits - m)
        o_ref[...] = e / jnp.sum(e, axis=1, keepdims=True)


def _tc_final(h, p, w, b, fw1, fb1, fw2p, fb2p, scale):
    return pl.pallas_call(
        _final_body,
        grid=(N // BM,),
        in_specs=[
            pl.BlockSpec(memory_space=pltpu.SMEM),
            pl.BlockSpec((BM, D), lambda i: (i, 0)),
            pl.BlockSpec((NC, BM, D), lambda i: (0, i, 0)),
            pl.BlockSpec((D, D), lambda i: (0, 0)),
            pl.BlockSpec((1, D), lambda i: (0, 0)),
            pl.BlockSpec((D, D), lambda i: (0, 0)),
            pl.BlockSpec((1, D), lambda i: (0, 0)),
            pl.BlockSpec((D, D), lambda i: (0, 0)),
            pl.BlockSpec((1, D), lambda i: (0, 0)),
        ],
        out_specs=pl.BlockSpec((1, D), lambda i: (0, 0)),
        out_shape=jax.ShapeDtypeStruct((1, D), jnp.float32),
        scratch_shapes=[pltpu.VMEM((1, D), jnp.float32)],
    )(scale, h, p, w, b, fw1, fb1, fw2p, fb2p)


def kernel(x, edge_index, eps, W0, b0, W1, b1, W2, b2, fcW1, fcb1, fcW2, fcb2):
    npad = EPAD - E
    src = jnp.concatenate([edge_index[0], jnp.zeros((npad,), jnp.int32)])
    pad_dst = N + (jnp.arange(npad, dtype=jnp.int32) % (NPAD - N))
    dst = jnp.concatenate([edge_index[1], pad_dst])
    srcp = src.reshape(NW, CH, K)
    dstp = dst.reshape(NW, CH, K)

    C = fcW2.shape[1]
    fw2p = jnp.zeros((D, D), jnp.float32).at[:, :C].set(fcW2)
    fb2p = jnp.full((1, D), -1e30, jnp.float32).at[0, :C].set(fcb2)

    b0r = b0.reshape(1, D)
    b1r = b1.reshape(1, D)
    b2r = b2.reshape(1, D)
    fb1r = fcb1.reshape(1, D)

    h = x
    layers = [(W0, b0r), (W1, b1r)]
    for i, (W, b) in enumerate(layers):
        p = _sc_agg(h, srcp, dstp)
        h = _tc_dense(h, p, W, b, (1.0 + eps[i]).reshape(1))
    p = _sc_agg(h, srcp, dstp)
    out = _tc_final(h, p, W2, b2r, fcW1, fb1r, fw2p, fb2p,
                    (1.0 + eps[2]).reshape(1))
    return out[0, :10]


# speedup vs baseline: 2.7810x; 2.7810x over previous
import functools

import jax
import jax.numpy as jnp
from jax import lax
from jax.experimental import pallas as pl
from jax.experimental.pallas import tpu as pltpu
from jax.experimental.pallas import tpu_sc as plsc

N = 10000
E = 320000
D = 128

NC = 2
NS = 16
NW = NC * NS

K = 125
CH = 80
EPAD = NW * CH * K
NPAD = 10240
RPT = NPAD // NS
ZB = 128


def _sc_agg_body(h_hbm, src_hbm, dst_hbm, out_hbm,
                 idx_src, idx_dst, rows0, pooled, sem0):
    cid = lax.axis_index("c")
    sid = lax.axis_index("s")
    wid = sid * NC + cid

    def _zrow(i, _):
        for j in range(D // 16):
            rows0[i, pl.ds(j * 16, 16)] = jnp.zeros((16,), jnp.float32)
        return 0
    lax.fori_loop(0, ZB, _zrow, 0)
    for t in range(RPT // ZB):
        pltpu.sync_copy(rows0, pooled.at[pl.ds(sid * RPT + t * ZB, ZB)])
    plsc.subcore_barrier()

    pltpu.sync_copy(src_hbm.at[wid], idx_src)
    pltpu.sync_copy(dst_hbm.at[wid], idx_dst)

    def _chunk(j, _):
        pltpu.async_copy(h_hbm.at[idx_src.at[j]],
                         rows0.at[pl.ds(0, K)], sem0).wait()
        pltpu.sync_copy(rows0.at[pl.ds(0, K)],
                        pooled.at[idx_dst.at[j]], add=True)
        return 0
    lax.fori_loop(0, CH, _chunk, 0)
    plsc.subcore_barrier()

    for t in range(RPT // ZB):
        r0 = sid * RPT + t * ZB
        pltpu.sync_copy(pooled.at[pl.ds(r0, ZB)], rows0)
        pltpu.sync_copy(rows0, out_hbm.at[cid, pl.ds(r0, ZB)])


@functools.lru_cache(maxsize=None)
def _make_sc_agg():
    return pl.kernel(
        _sc_agg_body,
        out_type=jax.ShapeDtypeStruct((NC, NPAD, D), jnp.float32),
        mesh=plsc.VectorSubcoreMesh(core_axis_name="c", subcore_axis_name="s",
                                    num_cores=NC, num_subcores=NS),
        scratch_types=[
            pltpu.VMEM((CH, K), jnp.int32),
            pltpu.VMEM((CH, K), jnp.int32),
            pltpu.VMEM((ZB, D), jnp.float32),
            pltpu.VMEM_SHARED((NPAD, D), jnp.float32),
            pltpu.SemaphoreType.DMA,
        ],
    )


def _sc_agg(h, srcp, dstp):
    return _make_sc_agg()(h, srcp, dstp)


BM = 1000


def _dense_body(s_ref, h_ref, p_ref, w_ref, b_ref, o_ref):
    t = s_ref[0] * h_ref[...] + jnp.sum(p_ref[...], axis=0)
    o_ref[...] = jnp.maximum(
        jnp.dot(t, w_ref[...], preferred_element_type=jnp.float32)
        + b_ref[...], 0.0)


def _tc_dense(h, p, w, b, scale):
    return pl.pallas_call(
        _dense_body,
        grid=(N // BM,),
        in_specs=[
            pl.BlockSpec(memory_space=pltpu.SMEM),
            pl.BlockSpec((BM, D), lambda i: (i, 0)),
            pl.BlockSpec((NC, BM, D), lambda i: (0, i, 0)),
            pl.BlockSpec((D, D), lambda i: (0, 0)),
            pl.BlockSpec((1, D), lambda i: (0, 0)),
        ],
        out_specs=pl.BlockSpec((BM, D), lambda i: (i, 0)),
        out_shape=jax.ShapeDtypeStruct((N, D), jnp.float32),
    )(scale, h, p, w, b)


def _final_body(s_ref, h_ref, p_ref, w_ref, b_ref,
                fw1_ref, fb1_ref, fw2_ref, fb2_ref, o_ref, acc):
    i = pl.program_id(0)

    @pl.when(i == 0)
    def _():
        acc[...] = jnp.zeros_like(acc)

    t = s_ref[0] * h_ref[...] + jnp.sum(p_ref[...], axis=0)
    t = jnp.maximum(
        jnp.dot(t, w_ref[...], preferred_element_type=jnp.float32)
        + b_ref[...], 0.0)
    acc[...] += jnp.sum(t, axis=0, keepdims=True)

    @pl.when(i == pl.num_programs(0) - 1)
    def _():
        g = acc[...]
        t1 = jnp.maximum(
            jnp.dot(g, fw1_ref[...], preferred_element_type=jnp.float32)
            + fb1_ref[...], 0.0)
        logits = (jnp.dot(t1, fw2_ref[...], preferred_element_type=jnp.float32)
                  + fb2_ref[...])
        m = jnp.max(logits, axis=1, keepdims=True)
        e = jnp.exp(logits - m)
        o_ref[...] = e / jnp.sum(e, axis=1, keepdims=True)


def _tc_final(h, p, w, b, fw1, fb1, fw2p, fb2p, scale):
    return pl.pallas_call(
        _final_body,
        grid=(N // BM,),
        in_specs=[
            pl.BlockSpec(memory_space=pltpu.SMEM),
            pl.BlockSpec((BM, D), lambda i: (i, 0)),
            pl.BlockSpec((NC, BM, D), lambda i: (0, i, 0)),
            pl.BlockSpec((D, D), lambda i: (0, 0)),
            pl.BlockSpec((1, D), lambda i: (0, 0)),
            pl.BlockSpec((D, D), lambda i: (0, 0)),
            pl.BlockSpec((1, D), lambda i: (0, 0)),
            pl.BlockSpec((D, D), lambda i: (0, 0)),
            pl.BlockSpec((1, D), lambda i: (0, 0)),
        ],
        out_specs=pl.BlockSpec((1, D), lambda i: (0, 0)),
        out_shape=jax.ShapeDtypeStruct((1, D), jnp.float32),
        scratch_shapes=[pltpu.VMEM((1, D), jnp.float32)],
    )(scale, h, p, w, b, fw1, fb1, fw2p, fb2p)


def kernel(x, edge_index, eps, W0, b0, W1, b1, W2, b2, fcW1, fcb1, fcW2, fcb2):
    npad = EPAD - E
    src = jnp.concatenate([edge_index[0], jnp.zeros((npad,), jnp.int32)])
    pad_dst = N + (jnp.arange(npad, dtype=jnp.int32) % (NPAD - N))
    dst = jnp.concatenate([edge_index[1], pad_dst])
    srcp = src.reshape(NW, CH, K)
    dstp = dst.reshape(NW, CH, K)

    C = fcW2.shape[1]
    fw2p = jnp.zeros((D, D), jnp.float32).at[:, :C].set(fcW2)
    fb2p = jnp.full((1, D), -1e30, jnp.float32).at[0, :C].set(fcb2)

    b0r = b0.reshape(1, D)
    b1r = b1.reshape(1, D)
    b2r = b2.reshape(1, D)
    fb1r = fcb1.reshape(1, D)

    h = x
    layers = [(W0, b0r), (W1, b1r)]
    for i, (W, b) in enumerate(layers):
        p = _sc_agg(h, srcp, dstp)
        h = _tc_dense(h, p, W, b, (1.0 + eps[i]).reshape(1))
    p = _sc_agg(h, srcp, dstp)
    out = _tc_final(h, p, W2, b2r, fcW1, fb1r, fw2p, fb2p,
                    (1.0 + eps[2]).reshape(1))
    return out[0, :10]
